# baseline (speedup 1.0000x reference)
import functools

import jax
import jax.numpy as jnp
from jax import lax
from jax.experimental import pallas as pl
from jax.experimental.pallas import tpu as pltpu
from jax.experimental.pallas import tpu_sc as plsc

S, D, H, DH, E, TOPK, F = 2048, 1024, 16, 64, 8, 2, 1024
HD = H * DH
EPS = 1e-5
THETA = 150000.0
BT = 256
NB = S * TOPK // BT + E
NPAD = NB * BT
NW = 32
TPW = S // NW
NEG = -1e9


def _qkv_body(x_ref, ln1_ref, wq_ref, wk_ref, wv_ref, q_ref, k_ref, v_ref):
    si = pl.program_id(0)
    bs = x_ref.shape[0]
    x = x_ref[...]
    xn = x * lax.rsqrt(jnp.mean(x * x, axis=-1, keepdims=True) + EPS) * ln1_ref[...]
    q = jnp.dot(xn, wq_ref[...], preferred_element_type=jnp.float32)
    k = jnp.dot(xn, wk_ref[...], preferred_element_type=jnp.float32)
    v = jnp.dot(xn, wv_ref[...], preferred_element_type=jnp.float32)
    pos = (lax.broadcasted_iota(jnp.int32, (bs, DH // 2), 0) + si * bs
           ).astype(jnp.float32)
    invf = jnp.exp(lax.broadcasted_iota(jnp.int32, (bs, DH // 2), 1)
                   .astype(jnp.float32) * (-jnp.log(THETA) / (DH // 2)))
    fr = pos * invf
    cosf, sinf = jnp.cos(fr), jnp.sin(fr)
    cos_full = jnp.concatenate([cosf] * (2 * H), axis=1)
    sin_full = jnp.concatenate([sinf] * (2 * H), axis=1)

    def rot_half(t):
        parts = []
        for o in range(0, HD, DH):
            parts.append(-t[:, o + DH // 2:o + DH])
            parts.append(t[:, o:o + DH // 2])
        return jnp.concatenate(parts, axis=1)

    q_ref[...] = q * cos_full + rot_half(q) * sin_full
    k_ref[...] = k * cos_full + rot_half(k) * sin_full
    v_ref[...] = v


def _qkv(x, ln1, wq, wk, wv):
    bs = 512
    grid = (S // bs,)
    blk = pl.BlockSpec((bs, D), lambda i: (i, 0))
    wspec = pl.BlockSpec((D, HD), lambda i: (0, 0))
    return pl.pallas_call(
        _qkv_body,
        grid=grid,
        in_specs=[blk, pl.BlockSpec((1, D), lambda i: (0, 0)), wspec, wspec, wspec],
        out_specs=[blk, blk, blk],
        out_shape=[jax.ShapeDtypeStruct((S, HD), jnp.float32)] * 3,
    )(x, ln1, wq, wk, wv)


def _attn_body(q_ref, k_ref, v_ref, o_ref):
    qi = pl.program_id(0)
    bq = q_ref.shape[0]
    q = q_ref[...]

    def compute(ncols):
        row = lax.broadcasted_iota(jnp.int32, (bq, ncols), 0) + qi * bq
        col = lax.broadcasted_iota(jnp.int32, (bq, ncols), 1)
        mask = jnp.where(col <= row, 0.0, NEG)
        outs = []
        for hh in range(H):
            o = hh * DH
            s = lax.dot_general(q[:, o:o + DH], k_ref[0:ncols, o:o + DH],
                                (((1,), (1,)), ((), ())),
                                preferred_element_type=jnp.float32) * (1.0 / 8.0)
            s = s + mask
            m = jnp.max(s, axis=-1, keepdims=True)
            p = jnp.exp(s - m)
            l = jnp.sum(p, axis=-1, keepdims=True)
            oh = jnp.dot(p, v_ref[0:ncols, o:o + DH],
                         preferred_element_type=jnp.float32)
            outs.append(oh / l)
        o_ref[...] = jnp.concatenate(outs, axis=1)

    nq = S // bq

    @pl.when(qi < nq // 4)
    def _():
        compute(S // 4)

    @pl.when((qi >= nq // 4) & (qi < nq // 2))
    def _():
        compute(S // 2)

    @pl.when((qi >= nq // 2) & (qi < 3 * nq // 4))
    def _():
        compute(3 * S // 4)

    @pl.when(qi >= 3 * nq // 4)
    def _():
        compute(S)


def _attention(q, k, v):
    bq = 256
    grid = (S // bq,)
    qspec = pl.BlockSpec((bq, HD), lambda i: (i, 0))
    kvspec = pl.BlockSpec((S, HD), lambda i: (0, 0))
    return pl.pallas_call(
        _attn_body,
        grid=grid,
        in_specs=[qspec, kvspec, kvspec],
        out_specs=qspec,
        out_shape=jax.ShapeDtypeStruct((S, HD), jnp.float32),
    )(q, k, v)


def _proj_body(ao_ref, res_ref, ln2_ref, wo_ref, rw_ref, h_ref, xn_ref, lt_ref):
    h = jnp.dot(ao_ref[...], wo_ref[...], preferred_element_type=jnp.float32) \
        + res_ref[...]
    h_ref[...] = h
    xn = h * lax.rsqrt(jnp.mean(h * h, axis=-1, keepdims=True) + EPS) * ln2_ref[...]
    xn_ref[...] = xn
    lt_ref[...] = lax.dot_general(rw_ref[...], xn, (((0,), (1,)), ((), ())),
                                  preferred_element_type=jnp.float32)


def _proj_router(ao, res, ln2, wo, rw):
    bs = 512
    grid = (S // bs,)
    blk = pl.BlockSpec((bs, D), lambda i: (i, 0))
    return pl.pallas_call(
        _proj_body,
        grid=grid,
        in_specs=[blk, blk, pl.BlockSpec((1, D), lambda i: (0, 0)),
                  pl.BlockSpec((HD, D), lambda i: (0, 0)),
                  pl.BlockSpec((D, E), lambda i: (0, 0))],
        out_specs=[blk, blk, pl.BlockSpec((E, bs), lambda i: (0, i))],
        out_shape=[jax.ShapeDtypeStruct((S, D), jnp.float32),
                   jax.ShapeDtypeStruct((S, D), jnp.float32),
                   jax.ShapeDtypeStruct((E, S), jnp.float32)],
    )(ao, res, ln2, wo, rw)


def _topk_body(lt_hbm, idx_hbm, gate_hbm, lbuf, i1b, i2b, g1b, g2b):
    wid = lax.axis_index("s") * 2 + lax.axis_index("c")
    base = wid * TPW
    for e in range(E):
        pltpu.sync_copy(lt_hbm.at[e, pl.ds(base, TPW)], lbuf.at[e])
    one_i = jnp.full((16,), 1, jnp.int32)
    zero_i = jnp.full((16,), 0, jnp.int32)
    zero_f = jnp.full((16,), 0.0, jnp.float32)
    big_f = jnp.full((16,), 1e30, jnp.float32)
    one_f = jnp.full((16,), 1.0, jnp.float32)
    evec = [jnp.full((16,), e, jnp.int32) for e in range(E)]

    def argmax8(lv):
        m = lv[0]
        for e in range(1, E):
            m = jnp.maximum(m, lv[e])
        idx = zero_i
        found = zero_i
        for e in range(E):
            isnew = jnp.where(lv[e] >= m, one_i, zero_i)
            hit = isnew * (one_i - found)
            idx = idx + evec[e] * hit
            found = jnp.maximum(found, isnew)
        return m, idx

    for g in range(TPW // 16):
        sl = pl.ds(g * 16, 16)
        lv = [lbuf[e, sl] for e in range(E)]
        m1, idx1 = argmax8(lv)
        lv2 = []
        for e in range(E):
            mk = jnp.where(idx1 == evec[e], big_f, zero_f)
            lv2.append(lv[e] - mk)
        m2, idx2 = argmax8(lv2)
        z = jnp.exp(lv[0] - m1)
        for e in range(1, E):
            z = z + jnp.exp(lv[e] - m1)
        i1b[sl] = idx1
        i2b[sl] = idx2
        g1b[sl] = one_f / z
        g2b[sl] = jnp.exp(m2 - m1) / z
    pltpu.sync_copy(i1b, idx_hbm.at[0, pl.ds(base, TPW)])
    pltpu.sync_copy(i2b, idx_hbm.at[1, pl.ds(base, TPW)])
    pltpu.sync_copy(g1b, gate_hbm.at[0, pl.ds(base, TPW)])
    pltpu.sync_copy(g2b, gate_hbm.at[1, pl.ds(base, TPW)])


def _sc_topk(lt):
    mesh = plsc.VectorSubcoreMesh(core_axis_name="c", subcore_axis_name="s")
    fn = pl.kernel(
        _topk_body,
        out_type=[jax.ShapeDtypeStruct((TOPK, S), jnp.int32),
                  jax.ShapeDtypeStruct((TOPK, S), jnp.float32)],
        mesh=mesh,
        scratch_types=[pltpu.VMEM((E, TPW), jnp.float32),
                       pltpu.VMEM((TPW,), jnp.int32),
                       pltpu.VMEM((TPW,), jnp.int32),
                       pltpu.VMEM((TPW,), jnp.float32),
                       pltpu.VMEM((TPW,), jnp.float32)],
    )
    return fn(lt)


def _rank_body(idx_ref, slot_ref, bexp_ref):
    idxf = idx_ref[...].astype(jnp.float32)
    r0 = lax.broadcasted_iota(jnp.int32, (S, S), 0)
    c0 = lax.broadcasted_iota(jnp.int32, (S, S), 1)
    ident = (r0 == c0).astype(jnp.float32)
    lower = (r0 > c0).astype(jnp.float32)
    idxc = lax.dot_general(ident, idxf, (((1,), (1,)), ((), ())),
                           preferred_element_type=jnp.float32)
    erow = lax.broadcasted_iota(jnp.int32, (S, E), 1).astype(jnp.float32)
    eq0 = (idxc[:, 0:1] == erow).astype(jnp.float32)
    eq1 = (idxc[:, 1:2] == erow).astype(jnp.float32)
    eq = eq0 + eq1
    ranks = jnp.dot(lower, eq, preferred_element_type=jnp.float32)
    counts = jnp.sum(eq, axis=0, keepdims=True)
    nblk = jnp.floor((counts + (BT - 1)) * (1.0 / BT))
    e8r = lax.broadcasted_iota(jnp.int32, (E, E), 0)
    e8c = lax.broadcasted_iota(jnp.int32, (E, E), 1)
    upper8 = (e8r < e8c).astype(jnp.float32)
    exclb = jnp.dot(nblk, upper8, preferred_element_type=jnp.float32)
    offs = exclb * BT
    s0 = jnp.sum(eq0 * (offs + ranks), axis=1, keepdims=True)
    s1 = jnp.sum(eq1 * (offs + ranks), axis=1, keepdims=True)
    slots = jnp.concatenate([s0, s1], axis=1)
    slot_t = lax.dot_general(slots, ident, (((0,), (0,)), ((), ())),
                             preferred_element_type=jnp.float32,
                             precision=lax.Precision.HIGHEST)
    slot_ref[...] = (slot_t + 0.5).astype(jnp.int32)
    inclb = exclb + nblk
    brow = lax.broadcasted_iota(jnp.int32, (NB, E), 0).astype(jnp.float32)
    bexp = jnp.sum((brow >= inclb).astype(jnp.float32), axis=1, keepdims=True)
    bexp_ref[...] = bexp.astype(jnp.int32)


def _rank(idx):
    return pl.pallas_call(
        _rank_body,
        grid=(1,),
        in_specs=[pl.BlockSpec((TOPK, S), lambda i: (0, 0))],
        out_specs=[pl.BlockSpec((TOPK, S), lambda i: (0, 0)),
                   pl.BlockSpec((NB, 1), lambda i: (0, 0))],
        out_shape=[jax.ShapeDtypeStruct((TOPK, S), jnp.int32),
                   jax.ShapeDtypeStruct((NB, 1), jnp.int32)],
    )(idx)


def _dispatch_body(xn_hbm, slot_hbm, gate_hbm, xg_hbm, sg_hbm,
                   rows, i1b, i2b, g1b, g2b, sem):
    wid = lax.axis_index("s") * 2 + lax.axis_index("c")
    base = wid * TPW
    d1 = pltpu.async_copy(slot_hbm.at[0, pl.ds(base, TPW)], i1b, sem)
    d2 = pltpu.async_copy(slot_hbm.at[1, pl.ds(base, TPW)], i2b, sem)
    d3 = pltpu.async_copy(gate_hbm.at[0, pl.ds(base, TPW)], g1b, sem)
    d4 = pltpu.async_copy(gate_hbm.at[1, pl.ds(base, TPW)], g2b, sem)
    d5 = pltpu.async_copy(xn_hbm.at[pl.ds(base, TPW)], rows, sem)
    d1.wait()
    d2.wait()
    d3.wait()
    d4.wait()
    d5.wait()
    c1 = pltpu.async_copy(rows, xg_hbm.at[i1b], sem)
    c2 = pltpu.async_copy(rows, xg_hbm.at[i2b], sem)
    c3 = pltpu.async_copy(g1b, sg_hbm.at[i1b], sem)
    c4 = pltpu.async_copy(g2b, sg_hbm.at[i2b], sem)
    c1.wait()
    c2.wait()
    c3.wait()
    c4.wait()


def _sc_dispatch(xn, slot, gate):
    mesh = plsc.VectorSubcoreMesh(core_axis_name="c", subcore_axis_name="s")
    fn = pl.kernel(
        _dispatch_body,
        out_type=[jax.ShapeDtypeStruct((NPAD, D), jnp.float32),
                  jax.ShapeDtypeStruct((NPAD,), jnp.float32)],
        mesh=mesh,
        scratch_types=[pltpu.VMEM((TPW, D), jnp.float32),
                       pltpu.VMEM((TPW,), jnp.int32),
                       pltpu.VMEM((TPW,), jnp.int32),
                       pltpu.VMEM((TPW,), jnp.float32),
                       pltpu.VMEM((TPW,), jnp.float32),
                       pltpu.SemaphoreType.DMA],
    )
    return fn(xn, slot, gate)


def _ffn_body(bexp_ref, xg_ref, sg_ref, wg_ref, wu_ref, wd_ref, yg_ref):
    b = pl.program_id(0)

    @pl.when(bexp_ref[b] < E)
    def _():
        x = xg_ref[...]
        g = jnp.dot(x, wg_ref[0], preferred_element_type=jnp.float32)
        u = jnp.dot(x, wu_ref[0], preferred_element_type=jnp.float32)
        a = (g / (1.0 + jnp.exp(-g))) * u
        y = jnp.dot(a, wd_ref[0], preferred_element_type=jnp.float32)
        yg_ref[...] = y * sg_ref[...]


def _ffn(bexp, xg, sgate, wg, wu, wd):
    emap = lambda b, be: (jnp.minimum(be[b], E - 1), 0, 0)
    gs = pltpu.PrefetchScalarGridSpec(
        num_scalar_prefetch=1,
        grid=(NB,),
        in_specs=[pl.BlockSpec((BT, D), lambda b, be: (b, 0)),
                  pl.BlockSpec((BT, 1), lambda b, be: (b, 0)),
                  pl.BlockSpec((1, D, F), emap),
                  pl.BlockSpec((1, D, F), emap),
                  pl.BlockSpec((1, F, D), emap)],
        out_specs=pl.BlockSpec((BT, D), lambda b, be: (b, 0)),
    )
    return pl.pallas_call(
        _ffn_body,
        grid_spec=gs,
        out_shape=jax.ShapeDtypeStruct((NPAD, D), jnp.float32),
    )(bexp, xg, sgate.reshape(NPAD, 1), wg, wu, wd)


def _combine_body(h_hbm, yg_hbm, slot_hbm, out_hbm,
                  hb, y1, y2, i1b, i2b, sem):
    wid = lax.axis_index("s") * 2 + lax.axis_index("c")
    csz = TPW // 2
    for c in range(2):
        base = wid * TPW + c * csz
        pltpu.sync_copy(slot_hbm.at[0, pl.ds(base, csz)], i1b)
        pltpu.sync_copy(slot_hbm.at[1, pl.ds(base, csz)], i2b)
        c1 = pltpu.async_copy(h_hbm.at[pl.ds(base, csz)], hb, sem)
        c2 = pltpu.async_copy(yg_hbm.at[i1b], y1, sem)
        c3 = pltpu.async_copy(yg_hbm.at[i2b], y2, sem)
        c1.wait()
        c2.wait()
        c3.wait()

        def token_body(t, _):
            def col_body(j, _):
                for u in range(4):
                    cs = pl.ds((j * 4 + u) * 16, 16)
                    plsc.addupdate(hb.at[t, cs], y1[t, cs] + y2[t, cs])
                return 0

            lax.fori_loop(0, D // 64, col_body, 0)
            return 0

        lax.fori_loop(0, csz, token_body, 0)
        pltpu.sync_copy(hb, out_hbm.at[pl.ds(base, csz)])


def _sc_combine(h, yg, slot):
    mesh = plsc.VectorSubcoreMesh(core_axis_name="c", subcore_axis_name="s")
    csz = TPW // 2
    fn = pl.kernel(
        _combine_body,
        out_type=jax.ShapeDtypeStruct((S, D), jnp.float32),
        mesh=mesh,
        scratch_types=[pltpu.VMEM((csz, D), jnp.float32),
                       pltpu.VMEM((csz, D), jnp.float32),
                       pltpu.VMEM((csz, D), jnp.float32),
                       pltpu.VMEM((csz,), jnp.int32),
                       pltpu.VMEM((csz,), jnp.int32),
                       pltpu.SemaphoreType.DMA],
    )
    return fn(h, yg, slot)


def kernel(hidden_states, attention_mask, position_ids, ln1_w, ln2_w,
           Wq, Wk, Wv, Wo, router_w, Wg, Wu, Wd):
    x = hidden_states.reshape(S, D)
    ln1 = ln1_w.reshape(1, D)
    ln2 = ln2_w.reshape(1, D)
    q, k, v = _qkv(x, ln1, Wq, Wk, Wv)
    ao = _attention(q, k, v)
    h, xn, lt = _proj_router(ao, x, ln2, Wo, router_w)
    idx, gate = _sc_topk(lt)
    slot, bexp = _rank(idx)
    xg, sgate = _sc_dispatch(xn, slot, gate)
    yg = _ffn(bexp.reshape(NB), xg, sgate, Wg, Wu, Wd)
    out = _sc_combine(h, yg, slot)
    return out.reshape(1, S, D)


# speedup vs baseline: 1.5834x; 1.5834x over previous
import functools

import jax
import jax.numpy as jnp
from jax import lax
from jax.experimental import pallas as pl
from jax.experimental.pallas import tpu as pltpu
from jax.experimental.pallas import tpu_sc as plsc

S, D, H, DH, E, TOPK, F = 2048, 1024, 16, 64, 8, 2, 1024
HD = H * DH
EPS = 1e-5
THETA = 150000.0
BT = 256
NB = S * TOPK // BT + E
NPAD = NB * BT
NW = 32
TPW = S // NW
NEG = -1e9


def _qkv_body(x_ref, ln1_ref, wq_ref, wk_ref, wv_ref, q_ref, k_ref, v_ref):
    si = pl.program_id(0)
    bs = x_ref.shape[0]
    x = x_ref[...]
    xn = x * lax.rsqrt(jnp.mean(x * x, axis=-1, keepdims=True) + EPS) * ln1_ref[...]
    q = jnp.dot(xn, wq_ref[...], preferred_element_type=jnp.float32)
    k = jnp.dot(xn, wk_ref[...], preferred_element_type=jnp.float32)
    v = jnp.dot(xn, wv_ref[...], preferred_element_type=jnp.float32)
    pos = (lax.broadcasted_iota(jnp.int32, (bs, DH // 2), 0) + si * bs
           ).astype(jnp.float32)
    invf = jnp.exp(lax.broadcasted_iota(jnp.int32, (bs, DH // 2), 1)
                   .astype(jnp.float32) * (-jnp.log(THETA) / (DH // 2)))
    fr = pos * invf
    cosf, sinf = jnp.cos(fr), jnp.sin(fr)
    cos_full = jnp.concatenate([cosf] * (2 * H), axis=1)
    sin_full = jnp.concatenate([sinf] * (2 * H), axis=1)

    def rot_half(t):
        parts = []
        for o in range(0, HD, DH):
            parts.append(-t[:, o + DH // 2:o + DH])
            parts.append(t[:, o:o + DH // 2])
        return jnp.concatenate(parts, axis=1)

    q_ref[...] = q * cos_full + rot_half(q) * sin_full
    k_ref[...] = k * cos_full + rot_half(k) * sin_full
    v_ref[...] = v


def _qkv(x, ln1, wq, wk, wv):
    bs = 512
    grid = (S // bs,)
    blk = pl.BlockSpec((bs, D), lambda i: (i, 0))
    wspec = pl.BlockSpec((D, HD), lambda i: (0, 0))
    return pl.pallas_call(
        _qkv_body,
        grid=grid,
        in_specs=[blk, pl.BlockSpec((1, D), lambda i: (0, 0)), wspec, wspec, wspec],
        out_specs=[blk, blk, blk],
        out_shape=[jax.ShapeDtypeStruct((S, HD), jnp.float32)] * 3,
    )(x, ln1, wq, wk, wv)


def _attn_body(q_ref, k_ref, v_ref, o_ref):
    qi = pl.program_id(0)
    bq = q_ref.shape[0]
    q = q_ref[...]
    k = k_ref[...]
    v = v_ref[...]
    row = lax.broadcasted_iota(jnp.int32, (bq, S), 0) + qi * bq
    col = lax.broadcasted_iota(jnp.int32, (bq, S), 1)
    mask = jnp.where(col <= row, 0.0, NEG)
    outs = []
    for hh in range(H):
        o = hh * DH
        s = lax.dot_general(q[:, o:o + DH], k[:, o:o + DH],
                            (((1,), (1,)), ((), ())),
                            preferred_element_type=jnp.float32) * (1.0 / 8.0)
        s = s + mask
        m = jnp.max(s, axis=-1, keepdims=True)
        p = jnp.exp(s - m)
        l = jnp.sum(p, axis=-1, keepdims=True)
        oh = jnp.dot(p, v[:, o:o + DH], preferred_element_type=jnp.float32)
        outs.append(oh / l)
    o_ref[...] = jnp.concatenate(outs, axis=1)


def _attention(q, k, v):
    bq = 256
    grid = (S // bq,)
    qspec = pl.BlockSpec((bq, HD), lambda i: (i, 0))
    kvspec = pl.BlockSpec((S, HD), lambda i: (0, 0))
    return pl.pallas_call(
        _attn_body,
        grid=grid,
        in_specs=[qspec, kvspec, kvspec],
        out_specs=qspec,
        out_shape=jax.ShapeDtypeStruct((S, HD), jnp.float32),
    )(q, k, v)


def _proj_body(ao_ref, res_ref, ln2_ref, wo_ref, rw_ref, h_ref, xn_ref, lt_ref):
    h = jnp.dot(ao_ref[...], wo_ref[...], preferred_element_type=jnp.float32) \
        + res_ref[...]
    h_ref[...] = h
    xn = h * lax.rsqrt(jnp.mean(h * h, axis=-1, keepdims=True) + EPS) * ln2_ref[...]
    xn_ref[...] = xn
    lt_ref[...] = lax.dot_general(rw_ref[...], xn, (((0,), (1,)), ((), ())),
                                  preferred_element_type=jnp.float32)


def _proj_router(ao, res, ln2, wo, rw):
    bs = 512
    grid = (S // bs,)
    blk = pl.BlockSpec((bs, D), lambda i: (i, 0))
    return pl.pallas_call(
        _proj_body,
        grid=grid,
        in_specs=[blk, blk, pl.BlockSpec((1, D), lambda i: (0, 0)),
                  pl.BlockSpec((HD, D), lambda i: (0, 0)),
                  pl.BlockSpec((D, E), lambda i: (0, 0))],
        out_specs=[blk, blk, pl.BlockSpec((E, bs), lambda i: (0, i))],
        out_shape=[jax.ShapeDtypeStruct((S, D), jnp.float32),
                   jax.ShapeDtypeStruct((S, D), jnp.float32),
                   jax.ShapeDtypeStruct((E, S), jnp.float32)],
    )(ao, res, ln2, wo, rw)


def _topk_body(lt_hbm, idx_hbm, gate_hbm, lbuf, i1b, i2b, g1b, g2b):
    wid = lax.axis_index("s") * 2 + lax.axis_index("c")
    base = wid * TPW
    for e in range(E):
        pltpu.sync_copy(lt_hbm.at[e, pl.ds(base, TPW)], lbuf.at[e])
    one_i = jnp.full((16,), 1, jnp.int32)
    zero_i = jnp.full((16,), 0, jnp.int32)
    zero_f = jnp.full((16,), 0.0, jnp.float32)
    big_f = jnp.full((16,), 1e30, jnp.float32)
    one_f = jnp.full((16,), 1.0, jnp.float32)
    evec = [jnp.full((16,), e, jnp.int32) for e in range(E)]

    def argmax8(lv):
        m = lv[0]
        for e in range(1, E):
            m = jnp.maximum(m, lv[e])
        idx = zero_i
        found = zero_i
        for e in range(E):
            isnew = jnp.where(lv[e] >= m, one_i, zero_i)
            hit = isnew * (one_i - found)
            idx = idx + evec[e] * hit
            found = jnp.maximum(found, isnew)
        return m, idx

    for g in range(TPW // 16):
        sl = pl.ds(g * 16, 16)
        lv = [lbuf[e, sl] for e in range(E)]
        m1, idx1 = argmax8(lv)
        lv2 = []
        for e in range(E):
            mk = jnp.where(idx1 == evec[e], big_f, zero_f)
            lv2.append(lv[e] - mk)
        m2, idx2 = argmax8(lv2)
        z = jnp.exp(lv[0] - m1)
        for e in range(1, E):
            z = z + jnp.exp(lv[e] - m1)
        i1b[sl] = idx1
        i2b[sl] = idx2
        g1b[sl] = one_f / z
        g2b[sl] = jnp.exp(m2 - m1) / z
    pltpu.sync_copy(i1b, idx_hbm.at[0, pl.ds(base, TPW)])
    pltpu.sync_copy(i2b, idx_hbm.at[1, pl.ds(base, TPW)])
    pltpu.sync_copy(g1b, gate_hbm.at[0, pl.ds(base, TPW)])
    pltpu.sync_copy(g2b, gate_hbm.at[1, pl.ds(base, TPW)])


def _sc_topk(lt):
    mesh = plsc.VectorSubcoreMesh(core_axis_name="c", subcore_axis_name="s")
    fn = pl.kernel(
        _topk_body,
        out_type=[jax.ShapeDtypeStruct((TOPK, S), jnp.int32),
                  jax.ShapeDtypeStruct((TOPK, S), jnp.float32)],
        mesh=mesh,
        scratch_types=[pltpu.VMEM((E, TPW), jnp.float32),
                       pltpu.VMEM((TPW,), jnp.int32),
                       pltpu.VMEM((TPW,), jnp.int32),
                       pltpu.VMEM((TPW,), jnp.float32),
                       pltpu.VMEM((TPW,), jnp.float32)],
    )
    return fn(lt)


def _rank_body(idx_ref, slot_ref, bexp_ref):
    idxf = idx_ref[...].astype(jnp.float32)
    r0 = lax.broadcasted_iota(jnp.int32, (S, S), 0)
    c0 = lax.broadcasted_iota(jnp.int32, (S, S), 1)
    ident = (r0 == c0).astype(jnp.float32)
    lower = (r0 > c0).astype(jnp.float32)
    idxc = lax.dot_general(ident, idxf, (((1,), (1,)), ((), ())),
                           preferred_element_type=jnp.float32)
    erow = lax.broadcasted_iota(jnp.int32, (S, E), 1).astype(jnp.float32)
    eq0 = (idxc[:, 0:1] == erow).astype(jnp.float32)
    eq1 = (idxc[:, 1:2] == erow).astype(jnp.float32)
    eq = eq0 + eq1
    ranks = jnp.dot(lower, eq, preferred_element_type=jnp.float32)
    counts = jnp.sum(eq, axis=0, keepdims=True)
    nblk = jnp.floor((counts + (BT - 1)) * (1.0 / BT))
    e8r = lax.broadcasted_iota(jnp.int32, (E, E), 0)
    e8c = lax.broadcasted_iota(jnp.int32, (E, E), 1)
    upper8 = (e8r < e8c).astype(jnp.float32)
    exclb = jnp.dot(nblk, upper8, preferred_element_type=jnp.float32)
    offs = exclb * BT
    s0 = jnp.sum(eq0 * (offs + ranks), axis=1, keepdims=True)
    s1 = jnp.sum(eq1 * (offs + ranks), axis=1, keepdims=True)
    slots = jnp.concatenate([s0, s1], axis=1)
    slot_t = lax.dot_general(slots, ident, (((0,), (0,)), ((), ())),
                             preferred_element_type=jnp.float32,
                             precision=lax.Precision.HIGHEST)
    slot_ref[...] = (slot_t + 0.5).astype(jnp.int32)
    inclb = exclb + nblk
    brow = lax.broadcasted_iota(jnp.int32, (NB, E), 0).astype(jnp.float32)
    bexp = jnp.sum((brow >= inclb).astype(jnp.float32), axis=1, keepdims=True)
    bexp_ref[...] = bexp.astype(jnp.int32)


def _rank(idx):
    return pl.pallas_call(
        _rank_body,
        grid=(1,),
        in_specs=[pl.BlockSpec((TOPK, S), lambda i: (0, 0))],
        out_specs=[pl.BlockSpec((TOPK, S), lambda i: (0, 0)),
                   pl.BlockSpec((NB, 1), lambda i: (0, 0))],
        out_shape=[jax.ShapeDtypeStruct((TOPK, S), jnp.int32),
                   jax.ShapeDtypeStruct((NB, 1), jnp.int32)],
    )(idx)


def _dispatch_body(xn_hbm, slot_hbm, gate_hbm, xg_hbm, sg_hbm,
                   rows, i1b, i2b, g1b, g2b, sem):
    wid = lax.axis_index("s") * 2 + lax.axis_index("c")
    base = wid * TPW
    d1 = pltpu.async_copy(slot_hbm.at[0, pl.ds(base, TPW)], i1b, sem)
    d2 = pltpu.async_copy(slot_hbm.at[1, pl.ds(base, TPW)], i2b, sem)
    d3 = pltpu.async_copy(gate_hbm.at[0, pl.ds(base, TPW)], g1b, sem)
    d4 = pltpu.async_copy(gate_hbm.at[1, pl.ds(base, TPW)], g2b, sem)
    d5 = pltpu.async_copy(xn_hbm.at[pl.ds(base, TPW)], rows, sem)
    d1.wait()
    d2.wait()
    d3.wait()
    d4.wait()
    d5.wait()
    c1 = pltpu.async_copy(rows, xg_hbm.at[i1b], sem)
    c2 = pltpu.async_copy(rows, xg_hbm.at[i2b], sem)
    c3 = pltpu.async_copy(g1b, sg_hbm.at[i1b], sem)
    c4 = pltpu.async_copy(g2b, sg_hbm.at[i2b], sem)
    c1.wait()
    c2.wait()
    c3.wait()
    c4.wait()


def _sc_dispatch(xn, slot, gate):
    mesh = plsc.VectorSubcoreMesh(core_axis_name="c", subcore_axis_name="s")
    fn = pl.kernel(
        _dispatch_body,
        out_type=[jax.ShapeDtypeStruct((NPAD, D), jnp.float32),
                  jax.ShapeDtypeStruct((NPAD,), jnp.float32)],
        mesh=mesh,
        scratch_types=[pltpu.VMEM((TPW, D), jnp.float32),
                       pltpu.VMEM((TPW,), jnp.int32),
                       pltpu.VMEM((TPW,), jnp.int32),
                       pltpu.VMEM((TPW,), jnp.float32),
                       pltpu.VMEM((TPW,), jnp.float32),
                       pltpu.SemaphoreType.DMA],
    )
    return fn(xn, slot, gate)


def _ffn_body(bexp_ref, xg_ref, sg_ref, wg_ref, wu_ref, wd_ref, yg_ref):
    b = pl.program_id(0)

    @pl.when(bexp_ref[b] < E)
    def _():
        x = xg_ref[...]
        g = jnp.dot(x, wg_ref[0], preferred_element_type=jnp.float32)
        u = jnp.dot(x, wu_ref[0], preferred_element_type=jnp.float32)
        a = (g / (1.0 + jnp.exp(-g))) * u
        y = jnp.dot(a, wd_ref[0], preferred_element_type=jnp.float32)
        yg_ref[...] = y * sg_ref[...]


def _ffn(bexp, xg, sgate, wg, wu, wd):
    emap = lambda b, be: (jnp.minimum(be[b], E - 1), 0, 0)
    gs = pltpu.PrefetchScalarGridSpec(
        num_scalar_prefetch=1,
        grid=(NB,),
        in_specs=[pl.BlockSpec((BT, D), lambda b, be: (b, 0)),
                  pl.BlockSpec((BT, 1), lambda b, be: (b, 0)),
                  pl.BlockSpec((1, D, F), emap),
                  pl.BlockSpec((1, D, F), emap),
                  pl.BlockSpec((1, F, D), emap)],
        out_specs=pl.BlockSpec((BT, D), lambda b, be: (b, 0)),
    )
    return pl.pallas_call(
        _ffn_body,
        grid_spec=gs,
        out_shape=jax.ShapeDtypeStruct((NPAD, D), jnp.float32),
    )(bexp, xg, sgate.reshape(NPAD, 1), wg, wu, wd)


def _combine_body(h_hbm, yg_hbm, slot_hbm, out_hbm,
                  hb, y1, y2, i1b, i2b, sem):
    wid = lax.axis_index("s") * 2 + lax.axis_index("c")
    csz = TPW // 2
    for c in range(2):
        base = wid * TPW + c * csz
        pltpu.sync_copy(slot_hbm.at[0, pl.ds(base, csz)], i1b)
        pltpu.sync_copy(slot_hbm.at[1, pl.ds(base, csz)], i2b)
        c1 = pltpu.async_copy(h_hbm.at[pl.ds(base, csz)], hb, sem)
        c2 = pltpu.async_copy(yg_hbm.at[i1b], y1, sem)
        c3 = pltpu.async_copy(yg_hbm.at[i2b], y2, sem)
        c1.wait()
        c2.wait()
        c3.wait()

        def token_body(t, _):
            def col_body(j, _):
                for u in range(4):
                    cs = pl.ds((j * 4 + u) * 16, 16)
                    plsc.addupdate(hb.at[t, cs], y1[t, cs] + y2[t, cs])
                return 0

            lax.fori_loop(0, D // 64, col_body, 0)
            return 0

        lax.fori_loop(0, csz, token_body, 0)
        pltpu.sync_copy(hb, out_hbm.at[pl.ds(base, csz)])


def _sc_combine(h, yg, slot):
    mesh = plsc.VectorSubcoreMesh(core_axis_name="c", subcore_axis_name="s")
    csz = TPW // 2
    fn = pl.kernel(
        _combine_body,
        out_type=jax.ShapeDtypeStruct((S, D), jnp.float32),
        mesh=mesh,
        scratch_types=[pltpu.VMEM((csz, D), jnp.float32),
                       pltpu.VMEM((csz, D), jnp.float32),
                       pltpu.VMEM((csz, D), jnp.float32),
                       pltpu.VMEM((csz,), jnp.int32),
                       pltpu.VMEM((csz,), jnp.int32),
                       pltpu.SemaphoreType.DMA],
    )
    return fn(h, yg, slot)


def kernel(hidden_states, attention_mask, position_ids, ln1_w, ln2_w,
           Wq, Wk, Wv, Wo, router_w, Wg, Wu, Wd):
    x = hidden_states.reshape(S, D)
    ln1 = ln1_w.reshape(1, D)
    ln2 = ln2_w.reshape(1, D)
    q, k, v = _qkv(x, ln1, Wq, Wk, Wv)
    ao = _attention(q, k, v)
    h, xn, lt = _proj_router(ao, x, ln2, Wo, router_w)
    idx, gate = _sc_topk(lt)
    slot, bexp = _rank(idx)
    xg, sgate = _sc_dispatch(xn, slot, gate)
    yg = _ffn(bexp.reshape(NB), xg, sgate, Wg, Wu, Wd)
    out = _sc_combine(h, yg, slot)
    return out.reshape(1, S, D)
